# baseline (device time: 26823 ns/iter reference)
import jax
import jax.numpy as jnp
from jax import lax
from jax.experimental import pallas as pl
from jax.experimental.pallas import tpu as pltpu

M = 1024
H = M // 2
Q = M // 4
LANES = 8
BUFS_PER_LANE = 4
SLOTS = 6


def kernel(x, w_mat):
    m, k = x.shape
    _, n = w_mat.shape
    assert m == M
    nc = n // LANES

    def body(x_ref, w_ref, out_ref, *scratch):
        lane_bufs = [
            scratch[i * BUFS_PER_LANE:(i + 1) * BUFS_PER_LANE]
            for i in range(LANES)
        ]
        send_sems, recv_sems = scratch[LANES * BUFS_PER_LANE:]

        d = lax.axis_index("i")
        p1 = d ^ 1
        p2 = 3 - d
        diag = p2 ^ 1
        bit = d >> 1
        low = d & 1

        barrier_sem = pltpu.get_barrier_semaphore()
        for nbr in (p1, p2):
            pl.semaphore_signal(
                barrier_sem, inc=1,
                device_id=(nbr,), device_id_type=pl.DeviceIdType.MESH,
            )
        pl.semaphore_wait(barrier_sem, 2)

        def gq_a(e):
            return ((e & 1) ^ (e >> 1)) * H + (e >> 1) * Q

        def gq_b(e):
            return (e >> 1) * H + (e & 1) * Q

        cfgs = []
        for lane in range(LANES):
            a = lane % 2 == 0
            s1 = p1 if a else p2
            s2 = p2 if a else p1
            gq = gq_a if a else gq_b
            s2_of_s1 = (3 - s1) if a else (s1 ^ 1)
            cfgs.append(dict(
                s1=s1, s2=s2,
                dp=p1 if a else p2,
                rp=p2 if a else p1,
                gq=gq, s2_of_s1=s2_of_s1,
                cols=slice(lane * nc, (lane + 1) * nc),
                sb=SLOTS * lane,
                bufs=lane_bufs[lane],
            ))

        def dot_rows(rows, c):
            return jnp.dot(x_ref[pl.ds(rows, Q), :], w_ref[:, c["cols"]],
                           preferred_element_type=jnp.float32)

        def rdma(src_rows, dst_rows, c, slot, partner):
            return pltpu.make_async_remote_copy(
                src_ref=out_ref.at[pl.ds(src_rows, Q), c["cols"]],
                dst_ref=out_ref.at[pl.ds(dst_rows, Q), c["cols"]],
                send_sem=send_sems.at[c["sb"] + slot],
                recv_sem=recv_sems.at[c["sb"] + slot],
                device_id=(partner,), device_id_type=pl.DeviceIdType.MESH,
            )

        def exchange(src, dst, partner, s):
            r = pltpu.make_async_remote_copy(
                src_ref=src, dst_ref=dst,
                send_sem=send_sems.at[s], recv_sem=recv_sems.at[s],
                device_id=(partner,), device_id_type=pl.DeviceIdType.MESH,
            )
            r.start()
            return r

        pend = []

        r1b = []
        for c in cfgs:
            h_send = c["bufs"][0]
            h_send[Q:2 * Q, :] = dot_rows(
                c["gq"](c["s2_of_s1"]), c).astype(jnp.bfloat16)
            r1b.append(exchange(h_send.at[pl.ds(Q, Q), :],
                                c["bufs"][1].at[pl.ds(Q, Q), :],
                                c["s1"], c["sb"] + 0))
        pend += r1b

        for c in cfgs:
            h_send = c["bufs"][0]
            h_send[0:Q, :] = dot_rows(c["gq"](c["s1"]), c).astype(jnp.bfloat16)

        r2 = []
        r1a = []
        for c, r in zip(cfgs, r1b):
            (h_send, h_recv, q_send, _) = c["bufs"]
            r.wait_recv()
            q_send[...] = (dot_rows(c["gq"](c["s2"]), c)
                           + h_recv[Q:2 * Q, :].astype(jnp.float32)
                           ).astype(jnp.bfloat16)
            r2.append(exchange(q_send, c["bufs"][3], c["s2"], c["sb"] + 2))
            r1a.append(exchange(h_send.at[pl.ds(0, Q), :],
                                c["bufs"][1].at[pl.ds(0, Q), :],
                                c["s1"], c["sb"] + 1))
        pend += r2
        pend += r1a

        ag = []
        for c, ra, r in zip(cfgs, r1a, r2):
            (_, h_recv, _, q_recv) = c["bufs"]
            ra.wait_recv()
            r.wait_recv()
            red = (dot_rows(c["gq"](d), c)
                   + h_recv[0:Q, :].astype(jnp.float32)
                   + q_recv[...].astype(jnp.float32))
            red = jnp.maximum(red, 0.0)
            mine = c["gq"](d)
            out_ref[pl.ds(mine, Q), c["cols"]] = red.astype(jnp.bfloat16)
            s_p1 = rdma(mine, mine, c, 3, p1)
            s_p1.start()
            s_p2 = rdma(mine, mine, c, 4, p2)
            s_p2.start()
            ag.append((s_p1, s_p2))
        pend += [s for pair in ag for s in pair]

        relays = []
        for c in cfgs:
            dp_rows = c["gq"](c["dp"])
            dp_slot = 3 if c["dp"] is p1 else 4
            rdma(dp_rows, dp_rows, c, dp_slot, c["dp"]).wait_recv()
            rs = rdma(dp_rows, dp_rows, c, 5, c["rp"])
            rs.start()
            relays.append(rs)
        pend += relays

        for c in cfgs:
            rp_rows = c["gq"](c["rp"])
            rp_slot = 3 if c["rp"] is p1 else 4
            rdma(rp_rows, rp_rows, c, rp_slot, c["rp"]).wait_recv()
            dg_rows = c["gq"](diag)
            rdma(dg_rows, dg_rows, c, 5, c["rp"]).wait_recv()

        for r in pend:
            r.wait_send()

    lane_scratch = [
        pltpu.VMEM((2 * Q, nc), jnp.bfloat16),
        pltpu.VMEM((2 * Q, nc), jnp.bfloat16),
        pltpu.VMEM((Q, nc), jnp.bfloat16),
        pltpu.VMEM((Q, nc), jnp.bfloat16),
    ]
    return pl.pallas_call(
        body,
        out_shape=jax.ShapeDtypeStruct((M, n), jnp.bfloat16),
        in_specs=[
            pl.BlockSpec(memory_space=pltpu.VMEM),
            pl.BlockSpec(memory_space=pltpu.VMEM),
        ],
        out_specs=pl.BlockSpec(memory_space=pltpu.VMEM),
        scratch_shapes=lane_scratch * LANES + [
            pltpu.SemaphoreType.DMA((SLOTS * LANES,)),
            pltpu.SemaphoreType.DMA((SLOTS * LANES,)),
        ],
        compiler_params=pltpu.CompilerParams(collective_id=0),
    )(x, w_mat)
